# baseline (speedup 1.0000x reference)
import jax
import jax.numpy as jnp
from jax import lax
from jax.experimental import pallas as pl
from jax.experimental.pallas import tpu as pltpu
from jax.experimental.pallas import tpu_sc as plsc

_N = 10000
_H = 128
_G = 256
_BM = 200
_NC = 2
_NS = 16
_NW = _NC * _NS
_NP = 10240
_CH = _NP // _NW
_GP = _G + 8


def _hops_kernel(adj_ref, feat_ref, w0_ref, b0_ref, w1_ref, b1_ref,
                 idxf_ref, o_ref, ss_ref, ssq_ref, cnt_ref,
                 y0_vmem, y1_vmem):
    h = pl.program_id(0)
    i = pl.program_id(1)

    @pl.when(jnp.logical_and(h == 0, i == 0))
    def _y0():
        y0_vmem[...] = jnp.dot(feat_ref[...], w0_ref[...],
                               preferred_element_type=jnp.float32)

    @pl.when(h == 0)
    def _hop1():
        t = jnp.dot(adj_ref[...], y0_vmem[...],
                    preferred_element_type=jnp.float32)
        t = jnp.maximum(t + b0_ref[...], 0.0)
        y1_vmem[pl.ds(pl.multiple_of(i * _BM, _BM), _BM), :] = jnp.dot(
            t, w1_ref[...], preferred_element_type=jnp.float32)

    @pl.when(h == 1)
    def _hop2():
        t = jnp.dot(adj_ref[...], y1_vmem[...],
                    preferred_element_type=jnp.float32)
        t = jnp.maximum(t + b1_ref[...], 0.0)
        o_ref[...] = t
        ps = jnp.sum(t, axis=0, keepdims=True)
        psq = jnp.sum(t * t, axis=0, keepdims=True)
        seg_ids = jax.lax.broadcasted_iota(jnp.int32, (_G, _BM), 0).astype(
            jnp.float32)
        onehot = jnp.where(
            seg_ids == jnp.broadcast_to(idxf_ref[0:1, :], (_G, _BM)),
            1.0, 0.0)
        pc = jnp.broadcast_to(jnp.sum(onehot, axis=1, keepdims=True),
                              (_G, 8))

        @pl.when(i == 0)
        def _init():
            ss_ref[...] = ps
            ssq_ref[...] = psq
            cnt_ref[...] = pc

        @pl.when(i != 0)
        def _acc():
            ss_ref[...] += ps
            ssq_ref[...] += psq
            cnt_ref[...] += pc


def _hops(adj, feat, w0, b0, w1, b1, idxf):
    return pl.pallas_call(
        _hops_kernel,
        grid=(2, _N // _BM),
        in_specs=[
            pl.BlockSpec((_BM, _N), lambda h, i: (i, 0)),
            pl.BlockSpec((_N, _H), lambda h, i: (0, 0)),
            pl.BlockSpec((_H, _H), lambda h, i: (0, 0)),
            pl.BlockSpec((1, _H), lambda h, i: (0, 0)),
            pl.BlockSpec((_H, _H), lambda h, i: (0, 0)),
            pl.BlockSpec((1, _H), lambda h, i: (0, 0)),
            pl.BlockSpec((8, _BM), lambda h, i: (i, 0)),
        ],
        out_specs=[
            pl.BlockSpec((_BM, _H), lambda h, i: (i * h, 0)),
            pl.BlockSpec((1, _H), lambda h, i: (0, 0)),
            pl.BlockSpec((1, _H), lambda h, i: (0, 0)),
            pl.BlockSpec((_G, 8), lambda h, i: (0, 0)),
        ],
        out_shape=[
            jax.ShapeDtypeStruct((_NP, _H), jnp.float32),
            jax.ShapeDtypeStruct((1, _H), jnp.float32),
            jax.ShapeDtypeStruct((1, _H), jnp.float32),
            jax.ShapeDtypeStruct((_G, 8), jnp.float32),
        ],
        scratch_shapes=[pltpu.VMEM((_N, _H), jnp.float32),
                        pltpu.VMEM((_N, _H), jnp.float32)],
    )(adj, feat, w0, b0, w1, b1, idxf)


def _sc_segsum_kernel(x_hbm, idx_hbm, zf_hbm, segsum_hbm,
                      rows_v, idx_v, accf, sem_c):
    cid = lax.axis_index("c")
    sid = lax.axis_index("s")
    wid = sid * _NC + cid
    base = wid * _CH

    @pl.when(sid == 0)
    def _init():
        pltpu.sync_copy(zf_hbm, accf)

    pltpu.sync_copy(x_hbm.at[pl.ds(base, _CH)], rows_v)
    pltpu.sync_copy(idx_hbm.at[pl.ds(base, _CH)], idx_v)
    plsc.subcore_barrier()

    descs = []
    for g in range(_CH // 16):
        iv = idx_v[pl.ds(g * 16, 16)]
        descs.append(pltpu.async_copy(
            rows_v.at[pl.ds(g * 16, 16)], accf.at[iv], sem_c, add=True))
    for d in descs:
        d.wait()
    plsc.subcore_barrier()

    rows_per_tile = _G // _NS
    out_base = cid * _G + sid * rows_per_tile
    pltpu.sync_copy(accf.at[pl.ds(sid * rows_per_tile, rows_per_tile)],
                    segsum_hbm.at[pl.ds(out_base, rows_per_tile)])


def _sc_segsum(x2p, idxp):
    zf = jnp.zeros((_GP, _H), jnp.float32)
    mesh = plsc.VectorSubcoreMesh(core_axis_name="c", subcore_axis_name="s")
    return pl.kernel(
        _sc_segsum_kernel,
        out_type=jax.ShapeDtypeStruct((_NC * _G, _H), jnp.float32),
        mesh=mesh,
        scratch_types=[
            pltpu.VMEM((_CH, _H), jnp.float32),
            pltpu.VMEM((_CH,), jnp.int32),
            pltpu.VMEM_SHARED((_GP, _H), jnp.float32),
            pltpu.SemaphoreType.DMA,
        ],
    )(x2p, idxp, zf)


def _finalize_kernel(seg_ref, cnt_ref, ss_ref, ssq_ref,
                     bn1g_ref, bn1b_ref, bn2g_ref, bn2b_ref,
                     fc1w_ref, fc1b_ref, fc2w_ref, fc2b_ref, o_ref):
    n = jnp.float32(_N)
    mean = ss_ref[...] / n
    var = ssq_ref[...] / n - mean * mean
    a = bn1g_ref[...] * jax.lax.rsqrt(var + 1e-5)
    c = bn1b_ref[...] - a * mean
    pooled_raw = seg_ref[0:_G, :] + seg_ref[_G:2 * _G, :]
    counts = cnt_ref[:, 0:1]
    pooled = a * pooled_raw + counts * c
    g = jnp.float32(_G)
    mean2 = jnp.sum(pooled, axis=0, keepdims=True) / g
    var2 = jnp.sum(pooled * pooled, axis=0, keepdims=True) / g - mean2 * mean2
    p = bn2g_ref[...] * (pooled - mean2) * jax.lax.rsqrt(var2 + 1e-5) \
        + bn2b_ref[...]
    h = jnp.maximum(jnp.dot(p, fc1w_ref[...],
                            preferred_element_type=jnp.float32)
                    + fc1b_ref[...], 0.0)
    o = jnp.dot(h, fc2w_ref[...], preferred_element_type=jnp.float32) \
        + fc2b_ref[...]
    m = jnp.max(o, axis=1, keepdims=True)
    e = jnp.exp(o - m)
    lse = jnp.log(jnp.sum(e, axis=1, keepdims=True)) + m
    o_ref[...] = o - lse


def kernel(adj, features, idx, W0, b0, W1, b1, bn1_g, bn1_b, bn2_g, bn2_b,
           fc1_W, fc1_b, fc2_W, fc2_b):
    row = lambda v: v.reshape(1, -1).astype(jnp.float32)
    idx32 = idx.astype(jnp.int32)
    nb = _N // _BM
    idxf = jnp.broadcast_to(
        idx32.astype(jnp.float32).reshape(nb, 1, _BM),
        (nb, 8, _BM)).reshape(nb * 8, _BM)
    x2p, ss, ssq, cnt = _hops(adj, features, W0, row(b0), W1, row(b1),
                              idxf)
    idxp = jnp.pad(idx32, (0, _NP - _N), constant_values=_G)
    segsum_p = _sc_segsum(x2p, idxp)
    out = pl.pallas_call(
        _finalize_kernel,
        out_shape=jax.ShapeDtypeStruct((_G, 16), jnp.float32),
    )(segsum_p, cnt, ss, ssq, row(bn1_g), row(bn1_b), row(bn2_g),
      row(bn2_b), fc1_W, row(fc1_b), fc2_W, row(fc2_b))
    return out


# speedup vs baseline: 1.0124x; 1.0124x over previous
import jax
import jax.numpy as jnp
from jax import lax
from jax.experimental import pallas as pl
from jax.experimental.pallas import tpu as pltpu
from jax.experimental.pallas import tpu_sc as plsc

_N = 10000
_H = 128
_G = 256
_BM = 400
_NC = 2
_NS = 16
_NW = _NC * _NS
_NP = 10240
_CH = _NP // _NW
_GP = _G + 8


def _hops_kernel(adj_ref, feat_ref, w0_ref, b0_ref, w1_ref, b1_ref,
                 idxf_ref, o_ref, ss_ref, ssq_ref, cnt_ref,
                 y0_vmem, y1_vmem):
    h = pl.program_id(0)
    i = pl.program_id(1)

    @pl.when(jnp.logical_and(h == 0, i == 0))
    def _y0():
        y0_vmem[...] = jnp.dot(feat_ref[...], w0_ref[...],
                               preferred_element_type=jnp.float32)

    @pl.when(h == 0)
    def _hop1():
        t = jnp.dot(adj_ref[...], y0_vmem[...],
                    preferred_element_type=jnp.float32)
        t = jnp.maximum(t + b0_ref[...], 0.0)
        y1_vmem[pl.ds(pl.multiple_of(i * _BM, _BM), _BM), :] = jnp.dot(
            t, w1_ref[...], preferred_element_type=jnp.float32)

    @pl.when(h == 1)
    def _hop2():
        t = jnp.dot(adj_ref[...], y1_vmem[...],
                    preferred_element_type=jnp.float32)
        t = jnp.maximum(t + b1_ref[...], 0.0)
        o_ref[...] = t
        ps = jnp.sum(t, axis=0, keepdims=True)
        psq = jnp.sum(t * t, axis=0, keepdims=True)
        seg_ids = jax.lax.broadcasted_iota(jnp.int32, (_G, _BM), 0).astype(
            jnp.float32)
        onehot = jnp.where(
            seg_ids == jnp.broadcast_to(idxf_ref[0:1, :], (_G, _BM)),
            1.0, 0.0)
        pc = jnp.broadcast_to(jnp.sum(onehot, axis=1, keepdims=True),
                              (_G, 8))

        @pl.when(i == 0)
        def _init():
            ss_ref[...] = ps
            ssq_ref[...] = psq
            cnt_ref[...] = pc

        @pl.when(i != 0)
        def _acc():
            ss_ref[...] += ps
            ssq_ref[...] += psq
            cnt_ref[...] += pc


def _hops(adj, feat, w0, b0, w1, b1, idxf):
    return pl.pallas_call(
        _hops_kernel,
        grid=(2, _N // _BM),
        in_specs=[
            pl.BlockSpec((_BM, _N), lambda h, i: (i, 0)),
            pl.BlockSpec((_N, _H), lambda h, i: (0, 0)),
            pl.BlockSpec((_H, _H), lambda h, i: (0, 0)),
            pl.BlockSpec((1, _H), lambda h, i: (0, 0)),
            pl.BlockSpec((_H, _H), lambda h, i: (0, 0)),
            pl.BlockSpec((1, _H), lambda h, i: (0, 0)),
            pl.BlockSpec((8, _BM), lambda h, i: (i, 0)),
        ],
        out_specs=[
            pl.BlockSpec((_BM, _H), lambda h, i: (i * h, 0)),
            pl.BlockSpec((1, _H), lambda h, i: (0, 0)),
            pl.BlockSpec((1, _H), lambda h, i: (0, 0)),
            pl.BlockSpec((_G, 8), lambda h, i: (0, 0)),
        ],
        out_shape=[
            jax.ShapeDtypeStruct((_NP, _H), jnp.float32),
            jax.ShapeDtypeStruct((1, _H), jnp.float32),
            jax.ShapeDtypeStruct((1, _H), jnp.float32),
            jax.ShapeDtypeStruct((_G, 8), jnp.float32),
        ],
        scratch_shapes=[pltpu.VMEM((_N, _H), jnp.float32),
                        pltpu.VMEM((_N, _H), jnp.float32)],
    )(adj, feat, w0, b0, w1, b1, idxf)


def _sc_segsum_kernel(x_hbm, idx_hbm, zf_hbm, segsum_hbm,
                      rows_v, idx_v, accf, sem_c):
    cid = lax.axis_index("c")
    sid = lax.axis_index("s")
    wid = sid * _NC + cid
    base = wid * _CH

    @pl.when(sid == 0)
    def _init():
        pltpu.sync_copy(zf_hbm, accf)

    pltpu.sync_copy(x_hbm.at[pl.ds(base, _CH)], rows_v)
    pltpu.sync_copy(idx_hbm.at[pl.ds(base, _CH)], idx_v)
    plsc.subcore_barrier()

    descs = []
    for g in range(_CH // 16):
        iv = idx_v[pl.ds(g * 16, 16)]
        descs.append(pltpu.async_copy(
            rows_v.at[pl.ds(g * 16, 16)], accf.at[iv], sem_c, add=True))
    for d in descs:
        d.wait()
    plsc.subcore_barrier()

    rows_per_tile = _G // _NS
    out_base = cid * _G + sid * rows_per_tile
    pltpu.sync_copy(accf.at[pl.ds(sid * rows_per_tile, rows_per_tile)],
                    segsum_hbm.at[pl.ds(out_base, rows_per_tile)])


def _sc_segsum(x2p, idxp):
    zf = jnp.zeros((_GP, _H), jnp.float32)
    mesh = plsc.VectorSubcoreMesh(core_axis_name="c", subcore_axis_name="s")
    return pl.kernel(
        _sc_segsum_kernel,
        out_type=jax.ShapeDtypeStruct((_NC * _G, _H), jnp.float32),
        mesh=mesh,
        scratch_types=[
            pltpu.VMEM((_CH, _H), jnp.float32),
            pltpu.VMEM((_CH,), jnp.int32),
            pltpu.VMEM_SHARED((_GP, _H), jnp.float32),
            pltpu.SemaphoreType.DMA,
        ],
    )(x2p, idxp, zf)


def _finalize_kernel(seg_ref, cnt_ref, ss_ref, ssq_ref,
                     bn1g_ref, bn1b_ref, bn2g_ref, bn2b_ref,
                     fc1w_ref, fc1b_ref, fc2w_ref, fc2b_ref, o_ref):
    n = jnp.float32(_N)
    mean = ss_ref[...] / n
    var = ssq_ref[...] / n - mean * mean
    a = bn1g_ref[...] * jax.lax.rsqrt(var + 1e-5)
    c = bn1b_ref[...] - a * mean
    pooled_raw = seg_ref[0:_G, :] + seg_ref[_G:2 * _G, :]
    counts = cnt_ref[:, 0:1]
    pooled = a * pooled_raw + counts * c
    g = jnp.float32(_G)
    mean2 = jnp.sum(pooled, axis=0, keepdims=True) / g
    var2 = jnp.sum(pooled * pooled, axis=0, keepdims=True) / g - mean2 * mean2
    p = bn2g_ref[...] * (pooled - mean2) * jax.lax.rsqrt(var2 + 1e-5) \
        + bn2b_ref[...]
    h = jnp.maximum(jnp.dot(p, fc1w_ref[...],
                            preferred_element_type=jnp.float32)
                    + fc1b_ref[...], 0.0)
    o = jnp.dot(h, fc2w_ref[...], preferred_element_type=jnp.float32) \
        + fc2b_ref[...]
    m = jnp.max(o, axis=1, keepdims=True)
    e = jnp.exp(o - m)
    lse = jnp.log(jnp.sum(e, axis=1, keepdims=True)) + m
    o_ref[...] = o - lse


def kernel(adj, features, idx, W0, b0, W1, b1, bn1_g, bn1_b, bn2_g, bn2_b,
           fc1_W, fc1_b, fc2_W, fc2_b):
    row = lambda v: v.reshape(1, -1).astype(jnp.float32)
    idx32 = idx.astype(jnp.int32)
    nb = _N // _BM
    idxf = jnp.broadcast_to(
        idx32.astype(jnp.float32).reshape(nb, 1, _BM),
        (nb, 8, _BM)).reshape(nb * 8, _BM)
    x2p, ss, ssq, cnt = _hops(adj, features, W0, row(b0), W1, row(b1),
                              idxf)
    idxp = jnp.pad(idx32, (0, _NP - _N), constant_values=_G)
    segsum_p = _sc_segsum(x2p, idxp)
    out = pl.pallas_call(
        _finalize_kernel,
        out_shape=jax.ShapeDtypeStruct((_G, 16), jnp.float32),
    )(segsum_p, cnt, ss, ssq, row(bn1_g), row(bn1_b), row(bn2_g),
      row(bn2_b), fc1_W, row(fc1_b), fc2_W, row(fc2_b))
    return out
